# Initial kernel scaffold; baseline (speedup 1.0000x reference)
import functools

import jax
import jax.numpy as jnp
from jax.experimental import pallas as pl


def _nn_stage(keys_ref, keys_t_ref, q_ref, pred_ref, mag_ref, elem_ref):
    q = q_ref[...]
    kt = keys_t_ref[...]
    qk = jax.lax.dot_general(
        q, kt, (((1,), (0,)), ((), ())), preferred_element_type=jnp.float32
    )
    b2 = jnp.sum(kt * kt, axis=0, keepdims=True)
    s = b2 - 2.0 * qk
    m = jnp.min(s, axis=1, keepdims=True)
    iota = jax.lax.broadcasted_iota(jnp.int32, s.shape, 1)
    cand = jnp.where(s == m, iota, jnp.int32(2 ** 30))
    idx = jnp.min(cand, axis=1, keepdims=True)
    onehot = (iota == idx).astype(jnp.float32)
    tgt = jax.lax.dot_general(
        onehot, keys_ref[...], (((1,), (0,)), ((), ())),
        preferred_element_type=jnp.float32,
    )
    toff = tgt - q
    mag_ref[...] = jnp.sqrt(jnp.sum(toff * toff, axis=1, keepdims=True))
    x = pred_ref[...] - toff
    ax = jnp.abs(x)
    e = jnp.where(ax < 1.0, 0.5 * x * x, ax - 0.5)
    elem_ref[...] = jnp.sum(e, axis=1, keepdims=True)


def _loss_stage(k_count, mag_ref, elem_ref, out_ref):
    mag = mag_ref[...]
    bits = jax.lax.bitcast_convert_type(mag, jnp.int32)

    def body(_, lohi):
        lo, hi = lohi
        mid = lo + (hi - lo) // 2
        cnt = jnp.sum((bits <= mid).astype(jnp.int32))
        take = cnt >= k_count
        return jnp.where(take, lo, mid + 1), jnp.where(take, mid, hi)

    _, hi = jax.lax.fori_loop(
        0, 31, body, (jnp.int32(0), jnp.int32(0x7F800000))
    )
    thresh = jax.lax.bitcast_convert_type(hi, jnp.float32)
    mask = (mag <= thresh).astype(jnp.float32)
    denom = jnp.maximum(jnp.sum(mask) * 3.0, 1.0)
    out_ref[0, 0] = jnp.sum(elem_ref[...] * mask) / denom


@jax.jit
def kernel(pred_off, queries, keys):
    Q, D = queries.shape
    C = keys.shape[0]
    QB = 512
    keys_t = keys.T

    mag, elem = pl.pallas_call(
        _nn_stage,
        grid=(Q // QB,),
        in_specs=[
            pl.BlockSpec((C, D), lambda i: (0, 0)),
            pl.BlockSpec((D, C), lambda i: (0, 0)),
            pl.BlockSpec((QB, D), lambda i: (i, 0)),
            pl.BlockSpec((QB, D), lambda i: (i, 0)),
        ],
        out_specs=[
            pl.BlockSpec((QB, 1), lambda i: (i, 0)),
            pl.BlockSpec((QB, 1), lambda i: (i, 0)),
        ],
        out_shape=[
            jax.ShapeDtypeStruct((Q, 1), jnp.float32),
            jax.ShapeDtypeStruct((Q, 1), jnp.float32),
        ],
    )(keys, keys_t, queries, pred_off)

    k_count = int(0.99 * (Q - 1)) + 1
    side = 128
    mag2d = mag.reshape(Q // side, side)
    elem2d = elem.reshape(Q // side, side)
    out = pl.pallas_call(
        functools.partial(_loss_stage, k_count),
        in_specs=[
            pl.BlockSpec((Q // side, side), lambda: (0, 0)),
            pl.BlockSpec((Q // side, side), lambda: (0, 0)),
        ],
        out_specs=pl.BlockSpec((1, 1), lambda: (0, 0)),
        out_shape=jax.ShapeDtypeStruct((1, 1), jnp.float32),
    )(mag2d, elem2d)
    return out[0, 0]


# speedup vs baseline: 1.1377x; 1.1377x over previous
import functools

import jax
import jax.numpy as jnp
from jax.experimental import pallas as pl


def _nn_stage(keys_ref, keys_t_ref, q_ref, pred_ref, mag_ref, elem_ref):
    q = q_ref[...]
    kt = keys_t_ref[...]
    qk = jax.lax.dot_general(
        q, kt, (((1,), (0,)), ((), ())), preferred_element_type=jnp.float32
    )
    b2 = jnp.sum(kt * kt, axis=0, keepdims=True)
    s = b2 - 2.0 * qk
    m = jnp.min(s, axis=1, keepdims=True)
    iota = jax.lax.broadcasted_iota(jnp.int32, s.shape, 1)
    cand = jnp.where(s == m, iota, jnp.int32(2 ** 30))
    idx = jnp.min(cand, axis=1, keepdims=True)
    onehot = (iota == idx).astype(jnp.float32)
    tgt = jax.lax.dot_general(
        onehot, keys_ref[...], (((1,), (0,)), ((), ())),
        preferred_element_type=jnp.float32,
    )
    toff = tgt - q
    mag_ref[...] = jnp.sqrt(jnp.sum(toff * toff, axis=1, keepdims=True))
    x = pred_ref[...] - toff
    ax = jnp.abs(x)
    e = jnp.where(ax < 1.0, 0.5 * x * x, ax - 0.5)
    elem_ref[...] = jnp.sum(e, axis=1, keepdims=True)


def _loss_stage(k_count, mag_ref, elem_ref, out_ref):
    mag = mag_ref[...]
    bits = jax.lax.bitcast_convert_type(mag, jnp.int32)

    def body(_, lohi):
        lo, hi = lohi
        mid = lo + (hi - lo) // 2
        cnt = jnp.sum((bits <= mid).astype(jnp.int32))
        take = cnt >= k_count
        return jnp.where(take, lo, mid + 1), jnp.where(take, mid, hi)

    _, hi = jax.lax.fori_loop(
        0, 31, body, (jnp.int32(0), jnp.int32(0x7F800000))
    )
    thresh = jax.lax.bitcast_convert_type(hi, jnp.float32)
    mask = (mag <= thresh).astype(jnp.float32)
    denom = jnp.maximum(jnp.sum(mask) * 3.0, 1.0)
    loss = jnp.sum(elem_ref[...] * mask) / denom
    out_ref[...] = jnp.broadcast_to(loss, (1, 1))


@jax.jit
def kernel(pred_off, queries, keys):
    Q, D = queries.shape
    C = keys.shape[0]
    QB = 512
    keys_t = keys.T

    mag, elem = pl.pallas_call(
        _nn_stage,
        grid=(Q // QB,),
        in_specs=[
            pl.BlockSpec((C, D), lambda i: (0, 0)),
            pl.BlockSpec((D, C), lambda i: (0, 0)),
            pl.BlockSpec((QB, D), lambda i: (i, 0)),
            pl.BlockSpec((QB, D), lambda i: (i, 0)),
        ],
        out_specs=[
            pl.BlockSpec((QB, 1), lambda i: (i, 0)),
            pl.BlockSpec((QB, 1), lambda i: (i, 0)),
        ],
        out_shape=[
            jax.ShapeDtypeStruct((Q, 1), jnp.float32),
            jax.ShapeDtypeStruct((Q, 1), jnp.float32),
        ],
    )(keys, keys_t, queries, pred_off)

    k_count = int(0.99 * (Q - 1)) + 1
    side = 128
    mag2d = mag.reshape(Q // side, side)
    elem2d = elem.reshape(Q // side, side)
    out = pl.pallas_call(
        functools.partial(_loss_stage, k_count),
        in_specs=[
            pl.BlockSpec((Q // side, side), lambda: (0, 0)),
            pl.BlockSpec((Q // side, side), lambda: (0, 0)),
        ],
        out_specs=pl.BlockSpec((1, 1), lambda: (0, 0)),
        out_shape=jax.ShapeDtypeStruct((1, 1), jnp.float32),
    )(mag2d, elem2d)
    return out[0, 0]


# speedup vs baseline: 3.3361x; 2.9323x over previous
import functools

import jax
import jax.numpy as jnp
from jax.experimental import pallas as pl
from jax.experimental.pallas import tpu as pltpu


def _nn_stage(n_sub, keys_ref, keys_t_ref, qt_ref, predt_ref,
              mag_ref, elem_ref):
    kt = keys_t_ref[...]
    keys = keys_ref[...]
    c = kt.shape[1]
    b2 = jnp.sum(kt * kt, axis=0, keepdims=True)
    k4 = jnp.concatenate([kt, b2], axis=0)
    k5 = jnp.concatenate(
        [keys, jnp.ones((c, 1), jnp.float32)], axis=1
    )
    qb = qt_ref.shape[1]
    sb = qb // n_sub
    for h in range(n_sub):
        cols = pl.ds(h * sb, sb)
        qt = qt_ref[:, cols]
        q4 = jnp.concatenate(
            [-2.0 * qt, jnp.ones((1, sb), jnp.float32)], axis=0
        )
        s = jax.lax.dot_general(
            k4, q4, (((0,), (0,)), ((), ())),
            preferred_element_type=jnp.float32,
        )
        m = jnp.min(s, axis=0, keepdims=True)
        onehot = (s == m).astype(jnp.float32)
        t4 = jax.lax.dot_general(
            k5, onehot, (((0,), (0,)), ((), ())),
            preferred_element_type=jnp.float32,
        )
        tgt = t4[0:3, :] / t4[3:4, :]
        toff = tgt - qt
        mag = jnp.sqrt(jnp.sum(toff * toff, axis=0, keepdims=True))
        x = predt_ref[:, cols] - toff
        ax = jnp.abs(x)
        e = jnp.where(ax < 1.0, 0.5 * x * x, ax - 0.5)
        elem = jnp.sum(e, axis=0, keepdims=True)
        lanes = mag_ref.shape[1]
        r0 = (h * sb) // lanes
        mag_ref[pl.ds(r0, sb // lanes), :] = mag.reshape(sb // lanes, lanes)
        elem_ref[pl.ds(r0, sb // lanes), :] = elem.reshape(sb // lanes, lanes)


def _loss_stage(k_count, mag_ref, elem_ref, out_ref):
    mag = mag_ref[...]
    bits = jax.lax.bitcast_convert_type(mag, jnp.int32)

    def body(_, lohi):
        lo, hi = lohi
        mid = lo + (hi - lo) // 2
        cnt = jnp.sum((bits <= mid).astype(jnp.int32))
        take = cnt >= k_count
        return jnp.where(take, lo, mid + 1), jnp.where(take, mid, hi)

    _, hi = jax.lax.fori_loop(
        0, 31, body, (jnp.int32(0), jnp.int32(0x7F800000))
    )
    thresh = jax.lax.bitcast_convert_type(hi, jnp.float32)
    mask = (mag <= thresh).astype(jnp.float32)
    denom = jnp.maximum(jnp.sum(mask) * 3.0, 1.0)
    loss = jnp.sum(elem_ref[...] * mask) / denom
    out_ref[...] = jnp.broadcast_to(loss, (1, 1))


@jax.jit
def kernel(pred_off, queries, keys):
    Q, D = queries.shape
    C = keys.shape[0]
    QB = 4096
    N_SUB = 8
    LANES = 512
    keys_t = keys.T
    queries_t = queries.T
    pred_t = pred_off.T
    rows_per_step = QB // LANES

    mag, elem = pl.pallas_call(
        functools.partial(_nn_stage, N_SUB),
        grid=(Q // QB,),
        in_specs=[
            pl.BlockSpec((C, D), lambda i: (0, 0)),
            pl.BlockSpec((D, C), lambda i: (0, 0)),
            pl.BlockSpec((D, QB), lambda i: (0, i)),
            pl.BlockSpec((D, QB), lambda i: (0, i)),
        ],
        out_specs=[
            pl.BlockSpec((rows_per_step, LANES), lambda i: (i, 0)),
            pl.BlockSpec((rows_per_step, LANES), lambda i: (i, 0)),
        ],
        out_shape=[
            jax.ShapeDtypeStruct((Q // LANES, LANES), jnp.float32),
            jax.ShapeDtypeStruct((Q // LANES, LANES), jnp.float32),
        ],
        compiler_params=pltpu.CompilerParams(
            dimension_semantics=("parallel",),
        ),
    )(keys, keys_t, queries_t, pred_t)

    k_count = int(0.99 * (Q - 1)) + 1
    out = pl.pallas_call(
        functools.partial(_loss_stage, k_count),
        in_specs=[
            pl.BlockSpec((Q // LANES, LANES), lambda: (0, 0)),
            pl.BlockSpec((Q // LANES, LANES), lambda: (0, 0)),
        ],
        out_specs=pl.BlockSpec((1, 1), lambda: (0, 0)),
        out_shape=jax.ShapeDtypeStruct((1, 1), jnp.float32),
    )(mag, elem)
    return out[0, 0]
